# baseline (speedup 1.0000x reference)
import functools

import jax
import jax.numpy as jnp
from jax import lax
from jax.experimental import pallas as pl
from jax.experimental.pallas import tpu as pltpu
from jax.experimental.pallas import tpu_sc as plsc

_NV = 8192
_D = 256
_NC = 2
_NS = 16
_RPT = _NV // (_NC * _NS)
_WPT = _RPT * _D

def _bf16_round(w16):
    u = lax.bitcast_convert_type(w16, jnp.int32)
    r = (u + 0x7FFF + (lax.shift_right_logical(u, 16) & 1)) & jnp.int32(-65536)
    return lax.bitcast_convert_type(r, jnp.float32)


def _sc_body(a_hbm, z_hbm, w1_hbm, w2_hbm,
             vals_o, idx_o, zq_o, loss_o,
             wv, zv, av, vb, vi, svals, sidx, lv, li,
             rowv, zqv):
    c = lax.axis_index("c")
    s = lax.axis_index("s")
    tile = c * _NS + s
    base_row = tile * _RPT

    pltpu.sync_copy(w1_hbm.at[pl.ds(base_row * _D, _WPT)], wv)
    pltpu.sync_copy(z_hbm, zv)
    pltpu.sync_copy(a_hbm, av)
    a = av[...][0]
    zch = [zv[pl.ds(16 * k, 16)] for k in range(16)]

    def row_body(r, carry):
        best, bidx = carry
        off = r * _D
        macc = jnp.zeros((16,), jnp.float32)
        bacc = jnp.zeros((16,), jnp.float32)
        for k in range(16):
            w16 = wv[pl.ds(off + 16 * k, 16)]
            wf = _bf16_round(w16)
            macc = macc + wf * zch[k]
            bacc = bacc + w16 * w16
        m = jnp.sum(macc)
        b = jnp.sum(bacc)
        d = (a + b) - 2.0 * m
        upd = d < best
        best = jnp.where(upd, d, best)
        bidx = jnp.where(upd, base_row + r, bidx)
        return best, bidx

    best, bidx = lax.fori_loop(
        0, _RPT, row_body,
        (jnp.float32(jnp.inf), jnp.int32(0)))

    vb[...] = jnp.full((16,), best, jnp.float32)
    vi[...] = jnp.full((16,), bidx, jnp.int32)
    pltpu.sync_copy(vb, svals.at[pl.ds(s * 16, 16)])
    pltpu.sync_copy(vi, sidx.at[pl.ds(s * 16, 16)])
    plsc.subcore_barrier()

    @pl.when(s == 0)
    def _merge():
        pltpu.sync_copy(svals, lv)
        pltpu.sync_copy(sidx, li)
        gmin = best
        widx = bidx
        for j in range(1, 16):
            vj = lv[pl.ds(16 * j, 16)][0]
            ij = li[pl.ds(16 * j, 16)][0]
            take = vj < gmin
            gmin = jnp.where(take, vj, gmin)
            widx = jnp.where(take, ij, widx)

        pltpu.sync_copy(w1_hbm.at[pl.ds(widx * _D, _D)], rowv)

        sacc = jnp.zeros((16,), jnp.float32)
        for k in range(16):
            rk = rowv[pl.ds(16 * k, 16)]
            zk = zv[pl.ds(16 * k, 16)]
            diff = rk - zk
            zqv[pl.ds(16 * k, 16)] = zk + diff
            sacc = sacc + diff * diff
        ssum = jnp.sum(sacc)
        loss = (ssum * jnp.float32(0.00390625)
                - ssum * jnp.float32(0.0009765625))

        pltpu.sync_copy(zqv, zq_o.at[c])
        vb[...] = jnp.full((16,), gmin, jnp.float32)
        pltpu.sync_copy(vb, vals_o.at[c])
        vi[...] = jnp.full((16,), widx, jnp.int32)
        pltpu.sync_copy(vi, idx_o.at[c])
        vb[...] = jnp.full((16,), loss, jnp.float32)
        pltpu.sync_copy(vb, loss_o.at[c])


_SC_CALL = functools.partial(
    pl.kernel,
    out_type=[
        jax.ShapeDtypeStruct((_NC, 16), jnp.float32),
        jax.ShapeDtypeStruct((_NC, 16), jnp.int32),
        jax.ShapeDtypeStruct((_NC, _D), jnp.float32),
        jax.ShapeDtypeStruct((_NC, 16), jnp.float32),
    ],
    mesh=plsc.VectorSubcoreMesh(core_axis_name="c", subcore_axis_name="s"),
    compiler_params=pltpu.CompilerParams(needs_layout_passes=False),
    scratch_types=[
        pltpu.VMEM((_WPT,), jnp.float32),
        pltpu.VMEM((_D,), jnp.float32),
        pltpu.VMEM((16,), jnp.float32),
        pltpu.VMEM((16,), jnp.float32),
        pltpu.VMEM((16,), jnp.int32),
        pltpu.VMEM_SHARED((256,), jnp.float32),
        pltpu.VMEM_SHARED((256,), jnp.int32),
        pltpu.VMEM((256,), jnp.float32),
        pltpu.VMEM((256,), jnp.int32),
        pltpu.VMEM((_D,), jnp.float32),
        pltpu.VMEM((_D,), jnp.float32),
    ],
)


def kernel(z, embedding_weight):
    latent_dim = embedding_weight.shape[1]
    z_flatten = jnp.transpose(z, (0, 2, 3, 1)).reshape(-1, latent_dim)
    a = jnp.sum(z_flatten ** 2, axis=-1, keepdims=True)
    a16 = jnp.broadcast_to(a.reshape(()), (16,))
    zf = z_flatten.reshape(latent_dim)
    w1 = embedding_weight.reshape(_NV * _D)

    vals, idxs, zqs, losses = _SC_CALL(_sc_body)(
        a16, zf, w1, embedding_weight)

    take1 = vals[1, 0] < vals[0, 0]
    idx = jnp.where(take1, idxs[1, 0], idxs[0, 0])
    zq = jnp.where(take1, zqs[1], zqs[0]).reshape(1, 1, latent_dim, 1)
    loss = jnp.where(take1, losses[1, 0], losses[0, 0])
    return (zq, idx, loss)


# speedup vs baseline: 1.5832x; 1.5832x over previous
import functools

import jax
import jax.numpy as jnp
from jax import lax
from jax.experimental import pallas as pl
from jax.experimental.pallas import tpu as pltpu
from jax.experimental.pallas import tpu_sc as plsc

_NV = 8192
_D = 256
_BLK = 4096
_NBLK = _NV // _BLK


def _scan_kernel(a_ref, zs_ref, w_ref, idx_ref, best_ref, bidx_ref):
    step = pl.program_id(0)

    @pl.when(step == 0)
    def _init():
        best_ref[0, 0] = jnp.float32(jnp.inf)
        bidx_ref[0, 0] = jnp.int32(0)

    w = w_ref[...]
    zs = zs_ref[...]
    a = a_ref[0, 0]

    wb = w.astype(jnp.bfloat16)
    m3 = lax.dot_general(zs, wb, (((1,), (1,)), ((), ())),
                         preferred_element_type=jnp.float32)
    m = (m3[0:1, :] + m3[1:2, :]) + m3[2:3, :]
    ones = jnp.ones((1, _D), jnp.bfloat16)
    sq = wb * wb
    b = lax.dot_general(ones, sq, (((1,), (1,)), ((), ())),
                        preferred_element_type=jnp.float32)
    d = (a + b) - 2.0 * m

    lmin = jnp.min(d)
    larg = jnp.argmin(d).astype(jnp.int32)

    @pl.when(lmin < best_ref[0, 0])
    def _update():
        best_ref[0, 0] = lmin
        bidx_ref[0, 0] = step * _BLK + larg

    @pl.when(step == _NBLK - 1)
    def _final():
        idx_ref[0, 0] = bidx_ref[0, 0]


def _lookup_body(i_hbm, z_hbm, w1_hbm, zq_o, loss_o,
                 iv, zv, rowv, zqv, vb):
    c = lax.axis_index("c")
    s = lax.axis_index("s")

    @pl.when((c == 0) & (s == 0))
    def _go():
        pltpu.sync_copy(i_hbm, iv)
        widx = iv[...][0]
        pltpu.sync_copy(z_hbm, zv)
        pltpu.sync_copy(w1_hbm.at[pl.ds(widx * _D, _D)], rowv)

        sacc = jnp.zeros((16,), jnp.float32)
        for k in range(16):
            rk = rowv[pl.ds(16 * k, 16)]
            zk = zv[pl.ds(16 * k, 16)]
            diff = rk - zk
            zqv[pl.ds(16 * k, 16)] = zk + diff
            sacc = sacc + diff * diff
        ssum = jnp.sum(sacc)
        loss = (ssum * jnp.float32(0.00390625)
                - ssum * jnp.float32(0.0009765625))

        pltpu.sync_copy(zqv, zq_o)
        vb[...] = jnp.full((16,), loss, jnp.float32)
        pltpu.sync_copy(vb, loss_o)


_LOOKUP_CALL = functools.partial(
    pl.kernel,
    out_type=[
        jax.ShapeDtypeStruct((_D,), jnp.float32),
        jax.ShapeDtypeStruct((16,), jnp.float32),
    ],
    mesh=plsc.VectorSubcoreMesh(core_axis_name="c", subcore_axis_name="s"),
    compiler_params=pltpu.CompilerParams(needs_layout_passes=False),
    scratch_types=[
        pltpu.VMEM((16,), jnp.int32),
        pltpu.VMEM((_D,), jnp.float32),
        pltpu.VMEM((_D,), jnp.float32),
        pltpu.VMEM((_D,), jnp.float32),
        pltpu.VMEM((16,), jnp.float32),
    ],
)


def kernel(z, embedding_weight):
    latent_dim = embedding_weight.shape[1]
    z_flatten = jnp.transpose(z, (0, 2, 3, 1)).reshape(-1, latent_dim)
    a = jnp.sum(z_flatten ** 2, axis=-1, keepdims=True)
    zh = z_flatten.astype(jnp.bfloat16)
    r1 = z_flatten - zh.astype(jnp.float32)
    zl = r1.astype(jnp.bfloat16)
    zl2 = (r1 - zl.astype(jnp.float32)).astype(jnp.bfloat16)
    zs = jnp.concatenate([zh, zl, zl2], axis=0)

    idx = pl.pallas_call(
        _scan_kernel,
        grid=(_NBLK,),
        in_specs=[
            pl.BlockSpec(memory_space=pltpu.SMEM),
            pl.BlockSpec((3, _D), lambda i: (0, 0)),
            pl.BlockSpec((_BLK, _D), lambda i: (i, 0)),
        ],
        out_specs=pl.BlockSpec(memory_space=pltpu.SMEM),
        out_shape=jax.ShapeDtypeStruct((1, 1), jnp.int32),
        scratch_shapes=[
            pltpu.SMEM((1, 1), jnp.float32),
            pltpu.SMEM((1, 1), jnp.int32),
        ],
    )(a, zs, embedding_weight)

    i16 = jnp.broadcast_to(idx.reshape(()), (16,))
    zf = z_flatten.reshape(latent_dim)
    w1 = embedding_weight.reshape(_NV * _D)

    zq_row, loss16 = _LOOKUP_CALL(_lookup_body)(i16, zf, w1)

    zq = zq_row.reshape(1, 1, latent_dim, 1)
    return (zq, idx.reshape(()), loss16[0])


# speedup vs baseline: 5.1914x; 3.2790x over previous
import jax
import jax.numpy as jnp
from jax import lax
from jax.experimental import pallas as pl
from jax.experimental.pallas import tpu as pltpu

_NV = 8192
_D = 256
_BLK = 4096
_NBLK = _NV // _BLK


def _vq_kernel(a_ref, z_ref, zs_ref, w_ref, wfull_ref,
               zq_ref, idx_ref, loss_ref,
               best_ref, bidx_ref, row_ref, sem):
    step = pl.program_id(0)

    @pl.when(step == 0)
    def _init():
        best_ref[0, 0] = jnp.float32(jnp.inf)
        bidx_ref[0, 0] = jnp.int32(0)

    w = w_ref[...]
    zs = zs_ref[...]
    a = a_ref[0, 0]

    wb = w.astype(jnp.bfloat16)
    m3 = lax.dot_general(zs, wb, (((1,), (1,)), ((), ())),
                         preferred_element_type=jnp.float32)
    m = (m3[0:1, :] + m3[1:2, :]) + m3[2:3, :]
    ones = jnp.ones((1, _D), jnp.bfloat16)
    sq = wb * wb
    b = lax.dot_general(ones, sq, (((1,), (1,)), ((), ())),
                        preferred_element_type=jnp.float32)
    d = (a + b) - 2.0 * m

    lmin = jnp.min(d)
    larg = jnp.argmin(d).astype(jnp.int32)

    @pl.when(lmin < best_ref[0, 0])
    def _update():
        best_ref[0, 0] = lmin
        bidx_ref[0, 0] = step * _BLK + larg

    @pl.when(step == _NBLK - 1)
    def _final():
        gidx = bidx_ref[0, 0]
        cp = pltpu.make_async_copy(
            wfull_ref.at[pl.ds(gidx, 1), :], row_ref, sem)
        cp.start()
        cp.wait()
        row = row_ref[...]
        z = z_ref[...]
        zq_ref[...] = z + (row - z)
        diff = row - z
        s = jnp.sum(diff * diff)
        idx_ref[0, 0] = gidx
        loss_ref[0, 0] = (s * jnp.float32(0.00390625)
                          - s * jnp.float32(0.0009765625))


def kernel(z, embedding_weight):
    latent_dim = embedding_weight.shape[1]
    z_flatten = jnp.transpose(z, (0, 2, 3, 1)).reshape(-1, latent_dim)
    a = jnp.sum(z_flatten ** 2, axis=-1, keepdims=True)
    zh = z_flatten.astype(jnp.bfloat16)
    r1 = z_flatten - zh.astype(jnp.float32)
    zl = r1.astype(jnp.bfloat16)
    zl2 = (r1 - zl.astype(jnp.float32)).astype(jnp.bfloat16)
    zs = jnp.concatenate([zh, zl, zl2], axis=0)

    zq_row, idx, loss = pl.pallas_call(
        _vq_kernel,
        grid=(_NBLK,),
        in_specs=[
            pl.BlockSpec(memory_space=pltpu.SMEM),
            pl.BlockSpec((1, _D), lambda i: (0, 0)),
            pl.BlockSpec((3, _D), lambda i: (0, 0)),
            pl.BlockSpec((_BLK, _D), lambda i: (i, 0)),
            pl.BlockSpec(memory_space=pltpu.MemorySpace.HBM),
        ],
        out_specs=[
            pl.BlockSpec((1, _D), lambda i: (0, 0)),
            pl.BlockSpec(memory_space=pltpu.SMEM),
            pl.BlockSpec(memory_space=pltpu.SMEM),
        ],
        out_shape=[
            jax.ShapeDtypeStruct((1, _D), jnp.float32),
            jax.ShapeDtypeStruct((1, 1), jnp.int32),
            jax.ShapeDtypeStruct((1, 1), jnp.float32),
        ],
        scratch_shapes=[
            pltpu.SMEM((1, 1), jnp.float32),
            pltpu.SMEM((1, 1), jnp.int32),
            pltpu.VMEM((1, _D), jnp.float32),
            pltpu.SemaphoreType.DMA,
        ],
    )(a, z_flatten, zs, embedding_weight, embedding_weight)

    zq = zq_row.reshape(1, 1, latent_dim, 1)
    return (zq, idx.reshape(()), loss.reshape(()))
